# baseline (speedup 1.0000x reference)
import functools

import jax
import jax.numpy as jnp
from jax import lax
from jax.experimental import pallas as pl
from jax.experimental.pallas import tpu as pltpu
from jax.experimental.pallas import tpu_sc as plsc

_mesh = plsc.ScalarSubcoreMesh(axis_name="c", num_cores=1)


@functools.partial(
    pl.kernel,
    mesh=_mesh,
    out_type=jax.ShapeDtypeStruct((1, 2, 64, 128), jnp.float32),
    scratch_types=[
        pltpu.SMEM((1,), jnp.int32),
        pltpu.VMEM_SHARED((1, 2, 64, 128), jnp.float32),
    ],
)
def _extract(table_hbm, step_hbm, out_hbm, step_s, row_sp):
    pltpu.sync_copy(step_hbm, step_s)
    s = step_s[0]
    pltpu.sync_copy(table_hbm.at[pl.ds(s, 1)], row_sp)
    pltpu.sync_copy(row_sp, out_hbm)


def kernel(table, step):
    step_vec = jnp.reshape(jnp.asarray(step, dtype=jnp.int32), (1,))
    tview = jnp.swapaxes(table, 2, 3)
    out = _extract(tview, step_vec)
    return jnp.swapaxes(out, 2, 3)


# speedup vs baseline: 1.0069x; 1.0069x over previous
import functools

import jax
import jax.numpy as jnp
from jax.experimental import pallas as pl
from jax.experimental.pallas import tpu as pltpu
from jax.experimental.pallas import tpu_sc as plsc

_mesh = plsc.ScalarSubcoreMesh(axis_name="c", num_cores=1)


@functools.partial(
    pl.kernel,
    mesh=_mesh,
    out_type=jax.ShapeDtypeStruct((1, 2, 64, 128), jnp.float32),
    scratch_types=[
        pltpu.SMEM((1,), jnp.int32),
        pltpu.VMEM_SHARED((1, 2, 64, 128), jnp.float32),
    ],
)
def _extract(table_hbm, step_hbm, out_hbm, step_s, row_sp):
    pltpu.sync_copy(step_hbm, step_s)
    s = step_s[0]
    pltpu.sync_copy(table_hbm.at[pl.ds(s, 1)], row_sp)
    pltpu.sync_copy(row_sp, out_hbm)


def kernel(table, step):
    step_vec = jnp.reshape(jnp.asarray(step, dtype=jnp.int32), (1,))
    tview = jnp.swapaxes(table, 2, 3)
    out = _extract(tview, step_vec)
    return jnp.swapaxes(out, 2, 3)
